# baseline (speedup 1.0000x reference)
import functools

import jax
import jax.numpy as jnp
from jax import lax
from jax.experimental import pallas as pl
from jax.experimental.pallas import tpu as pltpu
from jax.experimental.pallas import tpu_sc as plsc

MAX_SEQ_LEN = 8192
EMBED_DIM = 1024

_NUM_CORES = 2
_NUM_SUBCORES = 16
_NUM_STREAM_WORKERS = _NUM_CORES * (_NUM_SUBCORES - 1)

_T_SLAB = 128
_T_CHUNK = 16
_T_CHUNKS = _T_SLAB // _T_CHUNK
_T_NBUF = 6

_S_ROWS_PER_CORE = (MAX_SEQ_LEN - _NUM_STREAM_WORKERS * _T_SLAB) // _NUM_CORES
_S_CHUNK = 128
_S_CHUNKS = _S_ROWS_PER_CORE // _S_CHUNK
_S_NBUF = 3

_MESH = plsc.VectorSubcoreMesh(core_axis_name="c", subcore_axis_name="s")


def _pipeline(table_hbm, out_hbm, buf, sems, base, chunk, n_chunks, nbuf):
    in_sems = list(sems[:nbuf])
    out_sems = list(sems[nbuf:])

    def start_in(i):
        slot = i % nbuf
        return pltpu.async_copy(
            table_hbm.at[pl.ds(base + i * chunk, chunk), :],
            buf.at[slot],
            in_sems[slot],
        )

    def start_out(i):
        slot = i % nbuf
        return pltpu.async_copy(
            buf.at[slot],
            out_hbm.at[pl.ds(base + i * chunk, chunk), :],
            out_sems[slot],
        )

    in_dma = [None] * n_chunks
    out_dma = [None] * n_chunks
    for i in range(min(nbuf - 1, n_chunks)):
        in_dma[i] = start_in(i)
    for i in range(n_chunks):
        in_dma[i].wait()
        out_dma[i] = start_out(i)
        nxt = i + nbuf - 1
        if nxt < n_chunks:
            if i >= 1:
                out_dma[i - 1].wait()
            in_dma[nxt] = start_in(nxt)
    for i in range(max(0, n_chunks - nbuf), n_chunks):
        if out_dma[i] is not None:
            out_dma[i].wait()


@functools.partial(
    pl.kernel,
    mesh=_MESH,
    out_type=jax.ShapeDtypeStruct((MAX_SEQ_LEN, EMBED_DIM), jnp.float32),
    scratch_types=[
        pltpu.VMEM((_T_NBUF, _T_CHUNK, EMBED_DIM), jnp.float32),
        pltpu.VMEM_SHARED((_S_NBUF, _S_CHUNK, EMBED_DIM), jnp.float32),
    ]
    + [pltpu.SemaphoreType.DMA] * (2 * _T_NBUF + 2 * _S_NBUF),
)
def _pos_embed_copy(table_hbm, out_hbm, tbuf, sbuf, *sems):
    cid = lax.axis_index("c")
    sid = lax.axis_index("s")
    t_sems = sems[: 2 * _T_NBUF]
    s_sems = sems[2 * _T_NBUF :]

    @pl.when(sid != 0)
    def _stream_side():
        swid = (sid - 1) * _NUM_CORES + cid
        base = swid * _T_SLAB
        _pipeline(table_hbm, out_hbm, tbuf, t_sems, base, _T_CHUNK, _T_CHUNKS, _T_NBUF)

    @pl.when(sid == 0)
    def _spmem_side():
        base = _NUM_STREAM_WORKERS * _T_SLAB + cid * _S_ROWS_PER_CORE
        _pipeline(table_hbm, out_hbm, sbuf, s_sems, base, _S_CHUNK, _S_CHUNKS, _S_NBUF)


def kernel(x, pos_embed):
    del x
    return _pos_embed_copy(pos_embed)[None]


# speedup vs baseline: 1.0465x; 1.0465x over previous
import functools

import jax
import jax.numpy as jnp
from jax import lax
from jax.experimental import pallas as pl
from jax.experimental.pallas import tpu as pltpu
from jax.experimental.pallas import tpu_sc as plsc

MAX_SEQ_LEN = 8192
EMBED_DIM = 1024

_NUM_CORES = 2
_NUM_SUBCORES = 16
_NUM_WORKERS = _NUM_CORES * _NUM_SUBCORES
_ROWS_PER_WORKER = MAX_SEQ_LEN // _NUM_WORKERS
_CHUNK_ROWS = 16
_NUM_CHUNKS = _ROWS_PER_WORKER // _CHUNK_ROWS
_NBUF = 6

_MESH = plsc.VectorSubcoreMesh(core_axis_name="c", subcore_axis_name="s")


@functools.partial(
    pl.kernel,
    mesh=_MESH,
    out_type=jax.ShapeDtypeStruct((MAX_SEQ_LEN, EMBED_DIM), jnp.float32),
    scratch_types=[
        pltpu.VMEM((_NBUF, _CHUNK_ROWS, EMBED_DIM), jnp.float32),
    ]
    + [pltpu.SemaphoreType.DMA] * (2 * _NBUF),
)
def _pos_embed_copy(table_hbm, out_hbm, buf, *sems):
    wid = lax.axis_index("s") * _NUM_CORES + lax.axis_index("c")
    base = wid * _ROWS_PER_WORKER
    in_sems = list(sems[:_NBUF])
    out_sems = list(sems[_NBUF:])

    def start_in(i):
        slot = i % _NBUF
        return pltpu.async_copy(
            table_hbm.at[pl.ds(base + i * _CHUNK_ROWS, _CHUNK_ROWS), :],
            buf.at[slot],
            in_sems[slot],
        )

    def start_out(i):
        slot = i % _NBUF
        return pltpu.async_copy(
            buf.at[slot],
            out_hbm.at[pl.ds(base + i * _CHUNK_ROWS, _CHUNK_ROWS), :],
            out_sems[slot],
        )

    in_dma = [None] * _NUM_CHUNKS
    out_dma = [None] * _NUM_CHUNKS
    for i in range(_NBUF - 1):
        in_dma[i] = start_in(i)
    for i in range(_NUM_CHUNKS):
        in_dma[i].wait()
        out_dma[i] = start_out(i)
        nxt = i + _NBUF - 1
        if nxt < _NUM_CHUNKS:
            if i >= 1:
                out_dma[i - 1].wait()
            in_dma[nxt] = start_in(nxt)
    for i in range(max(0, _NUM_CHUNKS - _NBUF), _NUM_CHUNKS):
        if out_dma[i] is not None:
            out_dma[i].wait()


def kernel(x, pos_embed):
    del x
    return _pos_embed_copy(pos_embed)[None]


# speedup vs baseline: 1.0499x; 1.0032x over previous
import functools

import jax
import jax.numpy as jnp
from jax import lax
from jax.experimental import pallas as pl
from jax.experimental.pallas import tpu as pltpu
from jax.experimental.pallas import tpu_sc as plsc

MAX_SEQ_LEN = 8192
EMBED_DIM = 1024

_NUM_CORES = 2
_NUM_SUBCORES = 16
_NUM_WORKERS = _NUM_CORES * _NUM_SUBCORES
_ROWS_PER_WORKER = MAX_SEQ_LEN // _NUM_WORKERS
_CHUNK_ROWS = 16
_NUM_CHUNKS = _ROWS_PER_WORKER // _CHUNK_ROWS
_NBUF = 7

_MESH = plsc.VectorSubcoreMesh(core_axis_name="c", subcore_axis_name="s")


@functools.partial(
    pl.kernel,
    mesh=_MESH,
    out_type=jax.ShapeDtypeStruct((MAX_SEQ_LEN, EMBED_DIM), jnp.float32),
    scratch_types=[
        pltpu.VMEM((_NBUF, _CHUNK_ROWS, EMBED_DIM), jnp.float32),
    ]
    + [pltpu.SemaphoreType.DMA] * (2 * _NBUF),
)
def _pos_embed_copy(table_hbm, out_hbm, buf, *sems):
    wid = lax.axis_index("s") * _NUM_CORES + lax.axis_index("c")
    base = wid * _ROWS_PER_WORKER
    in_sems = list(sems[:_NBUF])
    out_sems = list(sems[_NBUF:])

    def start_in(i):
        slot = i % _NBUF
        return pltpu.async_copy(
            table_hbm.at[pl.ds(base + i * _CHUNK_ROWS, _CHUNK_ROWS), :],
            buf.at[slot],
            in_sems[slot],
        )

    def start_out(i):
        slot = i % _NBUF
        return pltpu.async_copy(
            buf.at[slot],
            out_hbm.at[pl.ds(base + i * _CHUNK_ROWS, _CHUNK_ROWS), :],
            out_sems[slot],
        )

    in_dma = [None] * _NUM_CHUNKS
    out_dma = [None] * _NUM_CHUNKS
    for i in range(_NBUF - 1):
        in_dma[i] = start_in(i)
    for i in range(_NUM_CHUNKS):
        in_dma[i].wait()
        out_dma[i] = start_out(i)
        nxt = i + _NBUF - 1
        if nxt < _NUM_CHUNKS:
            if i >= 1:
                out_dma[i - 1].wait()
            in_dma[nxt] = start_in(nxt)
    for i in range(max(0, _NUM_CHUNKS - _NBUF), _NUM_CHUNKS):
        if out_dma[i] is not None:
            out_dma[i].wait()


def kernel(x, pos_embed):
    del x
    return _pos_embed_copy(pos_embed)[None]
